# baseline (speedup 1.0000x reference)
import functools
import math

import jax
import jax.numpy as jnp
from jax import lax
from jax.experimental import pallas as pl
from jax.experimental.pallas import tpu as pltpu
from jax.experimental.pallas import tpu_sc as plsc

N = 10000
E = 160000
D_IN = 256
D_HID = 512
D_OUT = 256

NC = 2
NS = 16
L = 16
NW = NC * NS

NPAD = 10240
NPASS = 2
QN = NPAD // (NC * NPASS)
R = QN // NS
ACC_R = R + 8
DUMP = R
EC = E // NS
PACKB = 12
PMASK = (1 << PACKB) - 1
QPAD = QN
CAP1 = 12288
CAP2 = 163840
LB = 2048
K = 64

_NOTILE = pltpu.CompilerParams(use_tc_tiling_on_sc=False)


def _m8(x):
    return pl.multiple_of(x, 8)


def _sign_ok(x, lo, hi):
    u = (x - lo) | (hi - 1 - x)
    return 1 ^ ((u >> 31) & 1)


def _make_prep():
    mesh = plsc.VectorSubcoreMesh(
        core_axis_name="c", subcore_axis_name="s",
        num_cores=NC, num_subcores=NS)

    out_type = [
        jax.ShapeDtypeStruct((NW * NPASS * CAP1,), jnp.int32),
        jax.ShapeDtypeStruct((NW * NPASS * L,), jnp.int32),
        jax.ShapeDtypeStruct((NW * NPASS * CAP2,), jnp.int32),
        jax.ShapeDtypeStruct((NW * NPASS * L,), jnp.int32),
    ]
    scratch = [
        pltpu.VMEM((EC,), jnp.int32),
        pltpu.VMEM((EC,), jnp.int32),
        pltpu.VMEM((CAP1 + L,), jnp.int32),
        pltpu.VMEM((LB,), jnp.int32),
        pltpu.VMEM((CAP1 + L,), jnp.int32),
        pltpu.VMEM((L,), jnp.int32),
        pltpu.SemaphoreType.DMA,
    ]

    def body(src_hbm, dst_hbm, c1_hbm, n1_hbm, c2_hbm, n2_hbm,
             src_e, dst_e, ob1, lbuf, ob2, cntv, sem):
        cid = lax.axis_index("c")
        sid = lax.axis_index("s")
        pltpu.sync_copy(src_hbm.at[pl.ds(_m8(sid * EC), EC)], src_e)
        pltpu.sync_copy(dst_hbm.at[pl.ds(_m8(sid * EC), EC)], dst_e)

        for p in range(NPASS):
            lo = cid * (NPASS * QN) + p * QN
            r1 = (sid * NC + cid) * NPASS + p

            def grp(g, off):
                d16 = dst_e[pl.ds(g * L, L)]
                s16 = src_e[pl.ds(g * L, L)]
                ok = _sign_ok(d16, lo, lo + QN)
                pk = (s16 << PACKB) | ((d16 - lo) & PMASK)
                for i in range(L):
                    ob1[pl.ds(off, L)] = jnp.broadcast_to(pk[i], (L,))
                    off = off + ok[i]
                return off

            off = lax.fori_loop(0, EC // L, grp, jnp.int32(0))
            padv = jnp.full((L,), QPAD, jnp.int32)
            for j in range(LB // L):
                ob1[pl.ds(off + j * L, L)] = padv
            pltpu.sync_copy(ob1.at[pl.ds(0, CAP1)],
                            c1_hbm.at[pl.ds(_m8(r1 * CAP1), CAP1)])
            npad1 = ((off + LB - 1) >> 11) << 11
            cntv[pl.ds(0, L)] = jnp.broadcast_to(npad1, (L,))
            pltpu.sync_copy(cntv, n1_hbm.at[pl.ds(_m8(r1 * L), L)])
        plsc.subcore_barrier()

        for p in range(NPASS):
            rlo = sid * R
            r2 = (sid * NC + cid) * NPASS + p
            base2 = r2 * CAP2
            hoff = jnp.int32(0)
            for s in range(NS):
                r1 = (s * NC + cid) * NPASS + p
                pltpu.sync_copy(n1_hbm.at[pl.ds(_m8(r1 * L), L)], cntv)
                n1 = cntv[pl.ds(0, L)][0]

                def chunk(ch, off):
                    pltpu.sync_copy(
                        c1_hbm.at[pl.ds(_m8(r1 * CAP1 + ch * LB), LB)], lbuf)

                    def grp(g, off):
                        pk = lbuf[pl.ds(g * L, L)]
                        dq = pk & PMASK
                        ok = _sign_ok(dq, rlo, rlo + R)
                        for i in range(L):
                            ob2[pl.ds(off, L)] = jnp.broadcast_to(pk[i], (L,))
                            off = off + ok[i]
                        return off

                    return lax.fori_loop(0, LB // L, grp, off)

                off = lax.fori_loop(0, n1 >> 11, chunk, jnp.int32(0))
                dpad = jnp.full((L,), rlo + DUMP, jnp.int32)
                for j in range(K // L):
                    ob2[pl.ds(off + j * L, L)] = dpad
                nblk = (off + K - 1) >> 6

                def flush(b, ho):
                    pltpu.sync_copy(
                        ob2.at[pl.ds(b * K, K)],
                        c2_hbm.at[pl.ds(_m8(base2 + ho + b * K), K)])
                    return ho

                lax.fori_loop(0, nblk, flush, hoff)
                hoff = hoff + nblk * K
            cntv[pl.ds(0, L)] = jnp.broadcast_to(hoff, (L,))
            pltpu.sync_copy(cntv, n2_hbm.at[pl.ds(_m8(r2 * L), L)])

    return pl.kernel(body, out_type=out_type, mesh=mesh,
                     scratch_types=scratch, compiler_params=_NOTILE)


def _make_agg(D, with_counts):
    mesh = plsc.VectorSubcoreMesh(
        core_axis_name="c", subcore_axis_name="s",
        num_cores=NC, num_subcores=NS)

    out_type = [jax.ShapeDtypeStruct((NPAD, D), jnp.float32)]
    scratch = [
        pltpu.VMEM((LB,), jnp.int32),
        pltpu.VMEM((K,), jnp.int32),
        pltpu.VMEM((K,), jnp.int32),
        pltpu.VMEM((K, D), jnp.float32),
        pltpu.VMEM((ACC_R, D), jnp.float32),
        pltpu.VMEM((L,), jnp.int32),
        pltpu.SemaphoreType.DMA,
    ]
    if with_counts:
        out_type.append(jax.ShapeDtypeStruct((NPAD, L), jnp.float32))
        scratch.append(pltpu.VMEM((ACC_R, L), jnp.float32))

    def body(h_hbm, c2_hbm, n2_hbm, *rest):
        if with_counts:
            (y_hbm, cnt_hbm, lbuf, gidx, locb, rows, acc, cntv, sem,
             acnt) = rest
        else:
            (y_hbm, lbuf, gidx, locb, rows, acc, cntv, sem) = rest
        cid = lax.axis_index("c")
        sid = lax.axis_index("s")
        rlo = sid * R
        one = jnp.full((L,), 1.0, jnp.float32)
        zeros = jnp.zeros((L,), jnp.float32)
        for p in range(NPASS):
            r2 = (sid * NC + cid) * NPASS + p
            base2 = r2 * CAP2
            nbase = cid * (NPASS * QN) + p * QN + sid * R

            def zb(i, _):
                for j in range(D // L):
                    acc[i, pl.ds(j * L, L)] = zeros
                if with_counts:
                    acnt[i, pl.ds(0, L)] = zeros
                return 0

            lax.fori_loop(0, ACC_R, zb, jnp.int32(0))
            pltpu.sync_copy(n2_hbm.at[pl.ds(_m8(r2 * L), L)], cntv)
            n2 = cntv[pl.ds(0, L)][0]
            nsub = n2 >> 6
            nch = (n2 + LB - 1) >> 11

            def chunk(ch, _):
                pltpu.sync_copy(
                    c2_hbm.at[pl.ds(_m8(base2 + ch * LB), LB)], lbuf)
                d = nsub - ch * (LB // K) - (LB // K)
                m = (LB // K) + (d & (d >> 31))

                def sub(q, _):
                    for j in range(K // L):
                        pk = lbuf[pl.ds(q * K + j * L, L)]
                        gidx[pl.ds(j * L, L)] = pk >> PACKB
                        locb[pl.ds(j * L, L)] = (pk & PMASK) - rlo
                    pltpu.async_copy(h_hbm.at[gidx], rows, sem).wait()

                    def grpacc(g, _):
                        lv = locb[pl.ds(g * L, L)]
                        for i in range(L):
                            o = lv[i]
                            e = g * L + i
                            for j in range(D // L):
                                acc[o, pl.ds(j * L, L)] = (
                                    acc[o, pl.ds(j * L, L)]
                                    + rows[e, pl.ds(j * L, L)])
                            if with_counts:
                                acnt[o, pl.ds(0, L)] = (
                                    acnt[o, pl.ds(0, L)] + one)
                        return 0

                    lax.fori_loop(0, K // L, grpacc, jnp.int32(0))
                    return 0

                lax.fori_loop(0, m, sub, jnp.int32(0))
                return 0

            lax.fori_loop(0, nch, chunk, jnp.int32(0))
            pltpu.sync_copy(acc.at[pl.ds(0, R)],
                            y_hbm.at[pl.ds(_m8(nbase), R)])
            if with_counts:
                pltpu.sync_copy(acnt.at[pl.ds(0, R)],
                                cnt_hbm.at[pl.ds(_m8(nbase), R)])

    return pl.kernel(body, out_type=out_type, mesh=mesh,
                     scratch_types=scratch, compiler_params=_NOTILE)


def _dense_body(relu, F, x_ref, y_ref, cnt_ref, wself_ref,
                wnei_ref, bcat_ref, wq_ref, wk_ref, wv_ref, wo_ref,
                bq_ref, bk_ref, bv_ref, bo_ref, out_ref):
    c = jnp.maximum(cnt_ref[:, 0:1], 1.0)
    mean = y_ref[...] / c
    hcat = (jnp.dot(x_ref[...], wself_ref[...],
                    preferred_element_type=jnp.float32)
            + jnp.dot(mean, wnei_ref[...],
                      preferred_element_type=jnp.float32)
            + bcat_ref[...])
    ht = hcat[:, :F]
    hs = hcat[:, F:]
    bq = bq_ref[...]
    bk = bk_ref[...]
    bv = bv_ref[...]
    wq = wq_ref[...]
    wk = wk_ref[...]
    wv = wv_ref[...]
    dot = functools.partial(jnp.dot, preferred_element_type=jnp.float32)
    qt = dot(ht, wq) + bq
    qs = dot(hs, wq) + bq
    kt = dot(ht, wk) + bk
    ks = dot(hs, wk) + bk
    vt = dot(ht, wv) + bv
    vs = dot(hs, wv) + bv
    sc = 1.0 / math.sqrt(F)
    ltt = jnp.sum(qt * kt, axis=1, keepdims=True) * sc
    lts = jnp.sum(qt * ks, axis=1, keepdims=True) * sc
    lst = jnp.sum(qs * kt, axis=1, keepdims=True) * sc
    lss = jnp.sum(qs * ks, axis=1, keepdims=True) * sc
    mt = jnp.maximum(ltt, lts)
    ms = jnp.maximum(lst, lss)
    ett = jnp.exp(ltt - mt)
    ets = jnp.exp(lts - mt)
    est = jnp.exp(lst - ms)
    ess = jnp.exp(lss - ms)
    ot = (ett * vt + ets * vs) / (ett + ets)
    os_ = (est * vt + ess * vs) / (est + ess)
    o = dot(0.5 * (ot + os_), wo_ref[...]) + bo_ref[...]
    if relu:
        o = jnp.maximum(o, 0.0)
    out_ref[...] = o


def _make_dense(Din, F, relu, BN=1000):
    grid = (N // BN,)
    row = lambda i: (i, 0)
    full = lambda i: (0, 0)
    return pl.pallas_call(
        functools.partial(_dense_body, relu, F),
        grid=grid,
        in_specs=[
            pl.BlockSpec((BN, Din), row),
            pl.BlockSpec((BN, Din), row),
            pl.BlockSpec((BN, L), row),
            pl.BlockSpec((Din, 2 * F), full),
            pl.BlockSpec((Din, 2 * F), full),
            pl.BlockSpec((1, 2 * F), full),
            pl.BlockSpec((F, F), full),
            pl.BlockSpec((F, F), full),
            pl.BlockSpec((F, F), full),
            pl.BlockSpec((F, F), full),
            pl.BlockSpec((1, F), full),
            pl.BlockSpec((1, F), full),
            pl.BlockSpec((1, F), full),
            pl.BlockSpec((1, F), full),
        ],
        out_specs=pl.BlockSpec((BN, F), row),
        out_shape=jax.ShapeDtypeStruct((N, F), jnp.float32),
    )


_prep = functools.cache(_make_prep)
_agg_l1 = functools.cache(lambda: _make_agg(D_IN, True))
_agg_l2 = functools.cache(lambda: _make_agg(D_HID, False))
_dense_l1 = _make_dense(D_IN, D_HID, True)
_dense_l2 = _make_dense(D_HID, D_OUT, False)


def kernel(x, edge_index,
           l1_topo_Wself, l1_topo_Wnei, l1_topo_b,
           l1_seq_Wself, l1_seq_Wnei, l1_seq_b,
           l1_Wq, l1_Wk, l1_Wv, l1_bq, l1_bk, l1_bv, l1_Wo, l1_bo,
           l2_topo_Wself, l2_topo_Wnei, l2_topo_b,
           l2_seq_Wself, l2_seq_Wnei, l2_seq_b,
           l2_Wq, l2_Wk, l2_Wv, l2_bq, l2_bk, l2_bv, l2_Wo, l2_bo):
    src = edge_index[0]
    dst = edge_index[1]
    _c1, _n1, c2, n2 = _prep()(src, dst)

    y1, cnt = _agg_l1()(x, c2, n2)
    w1self = jnp.concatenate([l1_topo_Wself, l1_seq_Wself], axis=1)
    w1nei = jnp.concatenate([l1_topo_Wnei, l1_seq_Wnei], axis=1)
    b1cat = jnp.concatenate([l1_topo_b, l1_seq_b])[None, :]
    h = _dense_l1(x, y1[:N], cnt[:N], w1self, w1nei, b1cat,
                  l1_Wq, l1_Wk, l1_Wv, l1_Wo,
                  l1_bq[None, :], l1_bk[None, :], l1_bv[None, :],
                  l1_bo[None, :])

    y2 = _agg_l2()(h, c2, n2)
    if isinstance(y2, (list, tuple)):
        y2, = y2
    w2self = jnp.concatenate([l2_topo_Wself, l2_seq_Wself], axis=1)
    w2nei = jnp.concatenate([l2_topo_Wnei, l2_seq_Wnei], axis=1)
    b2cat = jnp.concatenate([l2_topo_b, l2_seq_b])[None, :]
    out = _dense_l2(h, y2[:N], cnt[:N], w2self, w2nei, b2cat,
                    l2_Wq, l2_Wk, l2_Wv, l2_Wo,
                    l2_bq[None, :], l2_bk[None, :], l2_bv[None, :],
                    l2_bo[None, :])
    return out


# speedup vs baseline: 1.0716x; 1.0716x over previous
import functools
import math

import jax
import jax.numpy as jnp
from jax import lax
from jax.experimental import pallas as pl
from jax.experimental.pallas import tpu as pltpu
from jax.experimental.pallas import tpu_sc as plsc

N = 10000
E = 160000
D_IN = 256
D_HID = 512
D_OUT = 256

NC = 2
NS = 16
L = 16
NW = NC * NS

NPAD = 10240
NPASS = 2
QN = NPAD // (NC * NPASS)
R = QN // NS
ACC_R = R + 8
DUMP = R
EC = E // NS
PACKB = 12
PMASK = (1 << PACKB) - 1
QPAD = QN
CAP1 = 12288
CAP2 = 163840
LB = 2048
K = 64

_NOTILE = pltpu.CompilerParams(use_tc_tiling_on_sc=False)


def _m8(x):
    return pl.multiple_of(x, 8)


def _sign_ok(x, lo, hi):
    u = (x - lo) | (hi - 1 - x)
    return 1 ^ ((u >> 31) & 1)


def _make_prep():
    mesh = plsc.VectorSubcoreMesh(
        core_axis_name="c", subcore_axis_name="s",
        num_cores=NC, num_subcores=NS)

    out_type = [
        jax.ShapeDtypeStruct((NW * NPASS * CAP1,), jnp.int32),
        jax.ShapeDtypeStruct((NW * NPASS * L,), jnp.int32),
        jax.ShapeDtypeStruct((NW * NPASS * CAP2,), jnp.int32),
        jax.ShapeDtypeStruct((NW * NPASS * L,), jnp.int32),
    ]
    scratch = [
        pltpu.VMEM((EC,), jnp.int32),
        pltpu.VMEM((EC,), jnp.int32),
        pltpu.VMEM((CAP1 + L,), jnp.int32),
        pltpu.VMEM((LB,), jnp.int32),
        pltpu.VMEM((CAP1 + L,), jnp.int32),
        pltpu.VMEM((L,), jnp.int32),
        pltpu.SemaphoreType.DMA,
    ]

    def body(src_hbm, dst_hbm, c1_hbm, n1_hbm, c2_hbm, n2_hbm,
             src_e, dst_e, ob1, lbuf, ob2, cntv, sem):
        cid = lax.axis_index("c")
        sid = lax.axis_index("s")
        pltpu.sync_copy(src_hbm.at[pl.ds(_m8(sid * EC), EC)], src_e)
        pltpu.sync_copy(dst_hbm.at[pl.ds(_m8(sid * EC), EC)], dst_e)

        for p in range(NPASS):
            lo = cid * (NPASS * QN) + p * QN
            r1 = (sid * NC + cid) * NPASS + p

            def grp(g, off):
                d16 = dst_e[pl.ds(g * L, L)]
                s16 = src_e[pl.ds(g * L, L)]
                ok = _sign_ok(d16, lo, lo + QN)
                pk = (s16 << PACKB) | ((d16 - lo) & PMASK)
                for i in range(L):
                    ob1[pl.ds(off, L)] = jnp.broadcast_to(pk[i], (L,))
                    off = off + ok[i]
                return off

            off = lax.fori_loop(0, EC // L, grp, jnp.int32(0))
            padv = jnp.full((L,), QPAD, jnp.int32)
            for j in range(LB // L):
                ob1[pl.ds(off + j * L, L)] = padv
            pltpu.sync_copy(ob1.at[pl.ds(0, CAP1)],
                            c1_hbm.at[pl.ds(_m8(r1 * CAP1), CAP1)])
            npad1 = ((off + LB - 1) >> 11) << 11
            cntv[pl.ds(0, L)] = jnp.broadcast_to(npad1, (L,))
            pltpu.sync_copy(cntv, n1_hbm.at[pl.ds(_m8(r1 * L), L)])
        plsc.subcore_barrier()

        for p in range(NPASS):
            rlo = sid * R
            r2 = (sid * NC + cid) * NPASS + p
            base2 = r2 * CAP2
            hoff = jnp.int32(0)
            for s in range(NS):
                r1 = (s * NC + cid) * NPASS + p
                pltpu.sync_copy(n1_hbm.at[pl.ds(_m8(r1 * L), L)], cntv)
                n1 = cntv[pl.ds(0, L)][0]

                def chunk(ch, off):
                    pltpu.sync_copy(
                        c1_hbm.at[pl.ds(_m8(r1 * CAP1 + ch * LB), LB)], lbuf)

                    def grp(g, off):
                        pk = lbuf[pl.ds(g * L, L)]
                        dq = pk & PMASK
                        ok = _sign_ok(dq, rlo, rlo + R)
                        for i in range(L):
                            ob2[pl.ds(off, L)] = jnp.broadcast_to(pk[i], (L,))
                            off = off + ok[i]
                        return off

                    return lax.fori_loop(0, LB // L, grp, off)

                off = lax.fori_loop(0, n1 >> 11, chunk, jnp.int32(0))
                dpad = jnp.full((L,), rlo + DUMP, jnp.int32)
                for j in range(K // L):
                    ob2[pl.ds(off + j * L, L)] = dpad
                nblk = (off + K - 1) >> 6

                def flush(b, ho):
                    pltpu.sync_copy(
                        ob2.at[pl.ds(b * K, K)],
                        c2_hbm.at[pl.ds(_m8(base2 + ho + b * K), K)])
                    return ho

                lax.fori_loop(0, nblk, flush, hoff)
                hoff = hoff + nblk * K
            cntv[pl.ds(0, L)] = jnp.broadcast_to(hoff, (L,))
            pltpu.sync_copy(cntv, n2_hbm.at[pl.ds(_m8(r2 * L), L)])

    return pl.kernel(body, out_type=out_type, mesh=mesh,
                     scratch_types=scratch, compiler_params=_NOTILE)


def _make_agg(D, with_counts):
    mesh = plsc.VectorSubcoreMesh(
        core_axis_name="c", subcore_axis_name="s",
        num_cores=NC, num_subcores=NS)

    out_type = [jax.ShapeDtypeStruct((NPAD * D,), jnp.float32)]
    scratch = [
        pltpu.VMEM((LB,), jnp.int32),
        pltpu.VMEM((K,), jnp.int32),
        pltpu.VMEM((K,), jnp.int32),
        pltpu.VMEM((K, D), jnp.float32),
        pltpu.VMEM((ACC_R * D,), jnp.float32),
        pltpu.VMEM((L,), jnp.int32),
        pltpu.SemaphoreType.DMA,
    ]
    if with_counts:
        out_type.append(jax.ShapeDtypeStruct((NPAD, L), jnp.float32))
        scratch.append(pltpu.VMEM((ACC_R, L), jnp.float32))

    def body(h_hbm, c2_hbm, n2_hbm, *rest):
        if with_counts:
            (y_hbm, cnt_hbm, lbuf, gidx, locb, rows, acc, cntv, sem,
             acnt) = rest
        else:
            (y_hbm, lbuf, gidx, locb, rows, acc, cntv, sem) = rest
        cid = lax.axis_index("c")
        sid = lax.axis_index("s")
        rlo = sid * R
        one = jnp.full((L,), 1.0, jnp.float32)
        zeros = jnp.zeros((L,), jnp.float32)
        for p in range(NPASS):
            r2 = (sid * NC + cid) * NPASS + p
            base2 = r2 * CAP2
            nbase = cid * (NPASS * QN) + p * QN + sid * R

            @plsc.parallel_loop(0, ACC_R * D // L, 1, unroll=8)
            def _zb(i):
                acc[pl.ds(i * L, L)] = zeros

            if with_counts:
                @plsc.parallel_loop(0, ACC_R, 1, unroll=8)
                def _zc(i):
                    acnt[i, pl.ds(0, L)] = zeros
            pltpu.sync_copy(n2_hbm.at[pl.ds(_m8(r2 * L), L)], cntv)
            n2 = cntv[pl.ds(0, L)][0]
            nsub = n2 >> 6
            nch = (n2 + LB - 1) >> 11

            def chunk(ch, _):
                pltpu.sync_copy(
                    c2_hbm.at[pl.ds(_m8(base2 + ch * LB), LB)], lbuf)
                d = nsub - ch * (LB // K) - (LB // K)
                m = (LB // K) + (d & (d >> 31))

                def sub(q, _):
                    for j in range(K // L):
                        pk = lbuf[pl.ds(q * K + j * L, L)]
                        gidx[pl.ds(j * L, L)] = pk >> PACKB
                        locb[pl.ds(j * L, L)] = (pk & PMASK) - rlo
                    pltpu.async_copy(h_hbm.at[gidx], rows, sem).wait()

                    def grpacc(g, _):
                        lv = locb[pl.ds(g * L, L)]
                        for i in range(L):
                            ob = lv[i] * D
                            e = g * L + i

                            @plsc.parallel_loop(0, D // L, 1, unroll=4)
                            def _pacc(j):
                                acc[pl.ds(ob + j * L, L)] = (
                                    acc[pl.ds(ob + j * L, L)]
                                    + rows[e, pl.ds(j * L, L)])

                            if with_counts:
                                o = lv[i]
                                acnt[o, pl.ds(0, L)] = (
                                    acnt[o, pl.ds(0, L)] + one)
                        return 0

                    lax.fori_loop(0, K // L, grpacc, jnp.int32(0))
                    return 0

                lax.fori_loop(0, m, sub, jnp.int32(0))
                return 0

            lax.fori_loop(0, nch, chunk, jnp.int32(0))
            pltpu.sync_copy(acc.at[pl.ds(0, R * D)],
                            y_hbm.at[pl.ds(_m8(nbase * D), R * D)])
            if with_counts:
                pltpu.sync_copy(acnt.at[pl.ds(0, R)],
                                cnt_hbm.at[pl.ds(_m8(nbase), R)])

    return pl.kernel(body, out_type=out_type, mesh=mesh,
                     scratch_types=scratch, compiler_params=_NOTILE)


def _dense_body(relu, F, x_ref, y_ref, cnt_ref, wself_ref,
                wnei_ref, bcat_ref, wq_ref, wk_ref, wv_ref, wo_ref,
                bq_ref, bk_ref, bv_ref, bo_ref, out_ref):
    c = jnp.maximum(cnt_ref[:, 0:1], 1.0)
    mean = y_ref[...] / c
    hcat = (jnp.dot(x_ref[...], wself_ref[...],
                    preferred_element_type=jnp.float32)
            + jnp.dot(mean, wnei_ref[...],
                      preferred_element_type=jnp.float32)
            + bcat_ref[...])
    ht = hcat[:, :F]
    hs = hcat[:, F:]
    bq = bq_ref[...]
    bk = bk_ref[...]
    bv = bv_ref[...]
    wq = wq_ref[...]
    wk = wk_ref[...]
    wv = wv_ref[...]
    dot = functools.partial(jnp.dot, preferred_element_type=jnp.float32)
    qt = dot(ht, wq) + bq
    qs = dot(hs, wq) + bq
    kt = dot(ht, wk) + bk
    ks = dot(hs, wk) + bk
    vt = dot(ht, wv) + bv
    vs = dot(hs, wv) + bv
    sc = 1.0 / math.sqrt(F)
    ltt = jnp.sum(qt * kt, axis=1, keepdims=True) * sc
    lts = jnp.sum(qt * ks, axis=1, keepdims=True) * sc
    lst = jnp.sum(qs * kt, axis=1, keepdims=True) * sc
    lss = jnp.sum(qs * ks, axis=1, keepdims=True) * sc
    mt = jnp.maximum(ltt, lts)
    ms = jnp.maximum(lst, lss)
    ett = jnp.exp(ltt - mt)
    ets = jnp.exp(lts - mt)
    est = jnp.exp(lst - ms)
    ess = jnp.exp(lss - ms)
    ot = (ett * vt + ets * vs) / (ett + ets)
    os_ = (est * vt + ess * vs) / (est + ess)
    o = dot(0.5 * (ot + os_), wo_ref[...]) + bo_ref[...]
    if relu:
        o = jnp.maximum(o, 0.0)
    out_ref[...] = o


def _make_dense(Din, F, relu, BN=1000):
    grid = (N // BN,)
    row = lambda i: (i, 0)
    full = lambda i: (0, 0)
    return pl.pallas_call(
        functools.partial(_dense_body, relu, F),
        grid=grid,
        in_specs=[
            pl.BlockSpec((BN, Din), row),
            pl.BlockSpec((BN, Din), row),
            pl.BlockSpec((BN, L), row),
            pl.BlockSpec((Din, 2 * F), full),
            pl.BlockSpec((Din, 2 * F), full),
            pl.BlockSpec((1, 2 * F), full),
            pl.BlockSpec((F, F), full),
            pl.BlockSpec((F, F), full),
            pl.BlockSpec((F, F), full),
            pl.BlockSpec((F, F), full),
            pl.BlockSpec((1, F), full),
            pl.BlockSpec((1, F), full),
            pl.BlockSpec((1, F), full),
            pl.BlockSpec((1, F), full),
        ],
        out_specs=pl.BlockSpec((BN, F), row),
        out_shape=jax.ShapeDtypeStruct((N, F), jnp.float32),
    )


_prep = functools.cache(_make_prep)
_agg_l1 = functools.cache(lambda: _make_agg(D_IN, True))
_agg_l2 = functools.cache(lambda: _make_agg(D_HID, False))
_dense_l1 = _make_dense(D_IN, D_HID, True)
_dense_l2 = _make_dense(D_HID, D_OUT, False)


def kernel(x, edge_index,
           l1_topo_Wself, l1_topo_Wnei, l1_topo_b,
           l1_seq_Wself, l1_seq_Wnei, l1_seq_b,
           l1_Wq, l1_Wk, l1_Wv, l1_bq, l1_bk, l1_bv, l1_Wo, l1_bo,
           l2_topo_Wself, l2_topo_Wnei, l2_topo_b,
           l2_seq_Wself, l2_seq_Wnei, l2_seq_b,
           l2_Wq, l2_Wk, l2_Wv, l2_bq, l2_bk, l2_bv, l2_Wo, l2_bo):
    src = edge_index[0]
    dst = edge_index[1]
    _c1, _n1, c2, n2 = _prep()(src, dst)

    y1, cnt = _agg_l1()(x, c2, n2)
    y1 = y1.reshape(NPAD, D_IN)
    w1self = jnp.concatenate([l1_topo_Wself, l1_seq_Wself], axis=1)
    w1nei = jnp.concatenate([l1_topo_Wnei, l1_seq_Wnei], axis=1)
    b1cat = jnp.concatenate([l1_topo_b, l1_seq_b])[None, :]
    h = _dense_l1(x, y1[:N], cnt[:N], w1self, w1nei, b1cat,
                  l1_Wq, l1_Wk, l1_Wv, l1_Wo,
                  l1_bq[None, :], l1_bk[None, :], l1_bv[None, :],
                  l1_bo[None, :])

    y2 = _agg_l2()(h, c2, n2)
    if isinstance(y2, (list, tuple)):
        y2, = y2
    y2 = y2.reshape(NPAD, D_HID)
    w2self = jnp.concatenate([l2_topo_Wself, l2_seq_Wself], axis=1)
    w2nei = jnp.concatenate([l2_topo_Wnei, l2_seq_Wnei], axis=1)
    b2cat = jnp.concatenate([l2_topo_b, l2_seq_b])[None, :]
    out = _dense_l2(h, y2[:N], cnt[:N], w2self, w2nei, b2cat,
                    l2_Wq, l2_Wk, l2_Wv, l2_Wo,
                    l2_bq[None, :], l2_bk[None, :], l2_bv[None, :],
                    l2_bo[None, :])
    return out


# speedup vs baseline: 1.0733x; 1.0017x over previous
import functools
import math

import jax
import jax.numpy as jnp
from jax import lax
from jax.experimental import pallas as pl
from jax.experimental.pallas import tpu as pltpu
from jax.experimental.pallas import tpu_sc as plsc

N = 10000
E = 160000
D_IN = 256
D_HID = 512
D_OUT = 256

NC = 2
NS = 16
L = 16
NW = NC * NS

NPAD = 10240
NPASS = 2
QN = NPAD // (NC * NPASS)
R = QN // NS
ACC_R = R + 8
DUMP = R
EC = E // NS
PACKB = 12
PMASK = (1 << PACKB) - 1
QPAD = QN
CAP1 = 12288
CAP2 = 163840
LB = 2048
K = 64

_NOTILE = pltpu.CompilerParams(use_tc_tiling_on_sc=False)


def _m8(x):
    return pl.multiple_of(x, 8)


def _sign_ok(x, lo, hi):
    u = (x - lo) | (hi - 1 - x)
    return 1 ^ ((u >> 31) & 1)


def _make_prep():
    mesh = plsc.VectorSubcoreMesh(
        core_axis_name="c", subcore_axis_name="s",
        num_cores=NC, num_subcores=NS)

    out_type = [
        jax.ShapeDtypeStruct((NW * NPASS * CAP1,), jnp.int32),
        jax.ShapeDtypeStruct((NW * NPASS * L,), jnp.int32),
        jax.ShapeDtypeStruct((NW * NPASS * CAP2,), jnp.int32),
        jax.ShapeDtypeStruct((NW * NPASS * L,), jnp.int32),
    ]
    scratch = [
        pltpu.VMEM((EC,), jnp.int32),
        pltpu.VMEM((EC,), jnp.int32),
        pltpu.VMEM((CAP1 + L,), jnp.int32),
        pltpu.VMEM((LB,), jnp.int32),
        pltpu.VMEM((CAP1 + L,), jnp.int32),
        pltpu.VMEM((L,), jnp.int32),
        pltpu.SemaphoreType.DMA,
    ]

    def body(src_hbm, dst_hbm, c1_hbm, n1_hbm, c2_hbm, n2_hbm,
             src_e, dst_e, ob1, lbuf, ob2, cntv, sem):
        cid = lax.axis_index("c")
        sid = lax.axis_index("s")
        pltpu.sync_copy(src_hbm.at[pl.ds(_m8(sid * EC), EC)], src_e)
        pltpu.sync_copy(dst_hbm.at[pl.ds(_m8(sid * EC), EC)], dst_e)

        for p in range(NPASS):
            lo = cid * (NPASS * QN) + p * QN
            r1 = (sid * NC + cid) * NPASS + p

            def grp(g, off):
                d16 = dst_e[pl.ds(g * L, L)]
                s16 = src_e[pl.ds(g * L, L)]
                ok = _sign_ok(d16, lo, lo + QN)
                pk = (s16 << PACKB) | ((d16 - lo) & PMASK)
                for i in range(L):
                    ob1[pl.ds(off, L)] = jnp.broadcast_to(pk[i], (L,))
                    off = off + ok[i]
                return off

            off = lax.fori_loop(0, EC // L, grp, jnp.int32(0))
            padv = jnp.full((L,), QPAD, jnp.int32)
            for j in range(LB // L):
                ob1[pl.ds(off + j * L, L)] = padv
            pltpu.sync_copy(ob1.at[pl.ds(0, CAP1)],
                            c1_hbm.at[pl.ds(_m8(r1 * CAP1), CAP1)])
            npad1 = ((off + LB - 1) >> 11) << 11
            cntv[pl.ds(0, L)] = jnp.broadcast_to(npad1, (L,))
            pltpu.sync_copy(cntv, n1_hbm.at[pl.ds(_m8(r1 * L), L)])
        plsc.subcore_barrier()

        for p in range(NPASS):
            rlo = sid * R
            r2 = (sid * NC + cid) * NPASS + p
            base2 = r2 * CAP2
            hoff = jnp.int32(0)
            for s in range(NS):
                r1 = (s * NC + cid) * NPASS + p
                pltpu.sync_copy(n1_hbm.at[pl.ds(_m8(r1 * L), L)], cntv)
                n1 = cntv[pl.ds(0, L)][0]

                def chunk(ch, off):
                    pltpu.sync_copy(
                        c1_hbm.at[pl.ds(_m8(r1 * CAP1 + ch * LB), LB)], lbuf)

                    def grp(g, off):
                        pk = lbuf[pl.ds(g * L, L)]
                        dq = pk & PMASK
                        ok = _sign_ok(dq, rlo, rlo + R)
                        for i in range(L):
                            ob2[pl.ds(off, L)] = jnp.broadcast_to(pk[i], (L,))
                            off = off + ok[i]
                        return off

                    return lax.fori_loop(0, LB // L, grp, off)

                off = lax.fori_loop(0, n1 >> 11, chunk, jnp.int32(0))
                dpad = jnp.full((L,), rlo + DUMP, jnp.int32)
                for j in range(K // L):
                    ob2[pl.ds(off + j * L, L)] = dpad
                nblk = (off + K - 1) >> 6

                def flush(b, ho):
                    pltpu.sync_copy(
                        ob2.at[pl.ds(b * K, K)],
                        c2_hbm.at[pl.ds(_m8(base2 + ho + b * K), K)])
                    return ho

                lax.fori_loop(0, nblk, flush, hoff)
                hoff = hoff + nblk * K
            cntv[pl.ds(0, L)] = jnp.broadcast_to(hoff, (L,))
            pltpu.sync_copy(cntv, n2_hbm.at[pl.ds(_m8(r2 * L), L)])

    return pl.kernel(body, out_type=out_type, mesh=mesh,
                     scratch_types=scratch, compiler_params=_NOTILE)


def _make_agg(D, with_counts):
    mesh = plsc.VectorSubcoreMesh(
        core_axis_name="c", subcore_axis_name="s",
        num_cores=NC, num_subcores=NS)

    out_type = [jax.ShapeDtypeStruct((NPAD * D,), jnp.float32)]
    scratch = [
        pltpu.VMEM((LB,), jnp.int32),
        pltpu.VMEM((K,), jnp.int32),
        pltpu.VMEM((K,), jnp.int32),
        pltpu.VMEM((K, D), jnp.float32),
        pltpu.VMEM((ACC_R * D,), jnp.float32),
        pltpu.VMEM((L,), jnp.int32),
        pltpu.SemaphoreType.DMA,
    ]
    if with_counts:
        out_type.append(jax.ShapeDtypeStruct((NPAD, L), jnp.float32))
        scratch.append(pltpu.VMEM((ACC_R, L), jnp.float32))

    def body(h_hbm, c2_hbm, n2_hbm, *rest):
        if with_counts:
            (y_hbm, cnt_hbm, lbuf, gidx, locb, rows, acc, cntv, sem,
             acnt) = rest
        else:
            (y_hbm, lbuf, gidx, locb, rows, acc, cntv, sem) = rest
        cid = lax.axis_index("c")
        sid = lax.axis_index("s")
        rlo = sid * R
        one = jnp.full((L,), 1.0, jnp.float32)
        zeros = jnp.zeros((L,), jnp.float32)
        for p in range(NPASS):
            r2 = (sid * NC + cid) * NPASS + p
            base2 = r2 * CAP2
            nbase = cid * (NPASS * QN) + p * QN + sid * R

            @plsc.parallel_loop(0, ACC_R * D // L, 1, unroll=8)
            def _zb(i):
                acc[pl.ds(i * L, L)] = zeros

            if with_counts:
                @plsc.parallel_loop(0, ACC_R, 1, unroll=8)
                def _zc(i):
                    acnt[i, pl.ds(0, L)] = zeros
            pltpu.sync_copy(n2_hbm.at[pl.ds(_m8(r2 * L), L)], cntv)
            n2 = cntv[pl.ds(0, L)][0]
            nsub = n2 >> 6
            nch = (n2 + LB - 1) >> 11

            def chunk(ch, _):
                pltpu.sync_copy(
                    c2_hbm.at[pl.ds(_m8(base2 + ch * LB), LB)], lbuf)
                d = nsub - ch * (LB // K) - (LB // K)
                m = (LB // K) + (d & (d >> 31))

                def sub(q, _):
                    for j in range(K // L):
                        pk = lbuf[pl.ds(q * K + j * L, L)]
                        gidx[pl.ds(j * L, L)] = pk >> PACKB
                        locb[pl.ds(j * L, L)] = (pk & PMASK) - rlo
                    pltpu.async_copy(h_hbm.at[gidx], rows, sem).wait()

                    def grpacc(g, _):
                        lv = locb[pl.ds(g * L, L)]
                        obs = [lv[i] * D for i in range(L)]

                        @plsc.parallel_loop(0, D // L, 1, unroll=2)
                        def _pacc(j):
                            col = j * L
                            for i in range(L):
                                acc[pl.ds(obs[i] + col, L)] = (
                                    acc[pl.ds(obs[i] + col, L)]
                                    + rows[g * L + i, pl.ds(col, L)])

                        if with_counts:
                            for i in range(L):
                                o = lv[i]
                                acnt[o, pl.ds(0, L)] = (
                                    acnt[o, pl.ds(0, L)] + one)
                        return 0

                    lax.fori_loop(0, K // L, grpacc, jnp.int32(0))
                    return 0

                lax.fori_loop(0, m, sub, jnp.int32(0))
                return 0

            lax.fori_loop(0, nch, chunk, jnp.int32(0))
            pltpu.sync_copy(acc.at[pl.ds(0, R * D)],
                            y_hbm.at[pl.ds(_m8(nbase * D), R * D)])
            if with_counts:
                pltpu.sync_copy(acnt.at[pl.ds(0, R)],
                                cnt_hbm.at[pl.ds(_m8(nbase), R)])

    return pl.kernel(body, out_type=out_type, mesh=mesh,
                     scratch_types=scratch, compiler_params=_NOTILE)


def _dense_body(relu, F, x_ref, y_ref, cnt_ref, wself_ref,
                wnei_ref, bcat_ref, wq_ref, wk_ref, wv_ref, wo_ref,
                bq_ref, bk_ref, bv_ref, bo_ref, out_ref):
    c = jnp.maximum(cnt_ref[:, 0:1], 1.0)
    mean = y_ref[...] / c
    hcat = (jnp.dot(x_ref[...], wself_ref[...],
                    preferred_element_type=jnp.float32)
            + jnp.dot(mean, wnei_ref[...],
                      preferred_element_type=jnp.float32)
            + bcat_ref[...])
    ht = hcat[:, :F]
    hs = hcat[:, F:]
    bq = bq_ref[...]
    bk = bk_ref[...]
    bv = bv_ref[...]
    wq = wq_ref[...]
    wk = wk_ref[...]
    wv = wv_ref[...]
    dot = functools.partial(jnp.dot, preferred_element_type=jnp.float32)
    qt = dot(ht, wq) + bq
    qs = dot(hs, wq) + bq
    kt = dot(ht, wk) + bk
    ks = dot(hs, wk) + bk
    vt = dot(ht, wv) + bv
    vs = dot(hs, wv) + bv
    sc = 1.0 / math.sqrt(F)
    ltt = jnp.sum(qt * kt, axis=1, keepdims=True) * sc
    lts = jnp.sum(qt * ks, axis=1, keepdims=True) * sc
    lst = jnp.sum(qs * kt, axis=1, keepdims=True) * sc
    lss = jnp.sum(qs * ks, axis=1, keepdims=True) * sc
    mt = jnp.maximum(ltt, lts)
    ms = jnp.maximum(lst, lss)
    ett = jnp.exp(ltt - mt)
    ets = jnp.exp(lts - mt)
    est = jnp.exp(lst - ms)
    ess = jnp.exp(lss - ms)
    ot = (ett * vt + ets * vs) / (ett + ets)
    os_ = (est * vt + ess * vs) / (est + ess)
    o = dot(0.5 * (ot + os_), wo_ref[...]) + bo_ref[...]
    if relu:
        o = jnp.maximum(o, 0.0)
    out_ref[...] = o


def _make_dense(Din, F, relu, BN=1000):
    grid = (N // BN,)
    row = lambda i: (i, 0)
    full = lambda i: (0, 0)
    return pl.pallas_call(
        functools.partial(_dense_body, relu, F),
        grid=grid,
        in_specs=[
            pl.BlockSpec((BN, Din), row),
            pl.BlockSpec((BN, Din), row),
            pl.BlockSpec((BN, L), row),
            pl.BlockSpec((Din, 2 * F), full),
            pl.BlockSpec((Din, 2 * F), full),
            pl.BlockSpec((1, 2 * F), full),
            pl.BlockSpec((F, F), full),
            pl.BlockSpec((F, F), full),
            pl.BlockSpec((F, F), full),
            pl.BlockSpec((F, F), full),
            pl.BlockSpec((1, F), full),
            pl.BlockSpec((1, F), full),
            pl.BlockSpec((1, F), full),
            pl.BlockSpec((1, F), full),
        ],
        out_specs=pl.BlockSpec((BN, F), row),
        out_shape=jax.ShapeDtypeStruct((N, F), jnp.float32),
    )


_prep = functools.cache(_make_prep)
_agg_l1 = functools.cache(lambda: _make_agg(D_IN, True))
_agg_l2 = functools.cache(lambda: _make_agg(D_HID, False))
_dense_l1 = _make_dense(D_IN, D_HID, True)
_dense_l2 = _make_dense(D_HID, D_OUT, False)


def kernel(x, edge_index,
           l1_topo_Wself, l1_topo_Wnei, l1_topo_b,
           l1_seq_Wself, l1_seq_Wnei, l1_seq_b,
           l1_Wq, l1_Wk, l1_Wv, l1_bq, l1_bk, l1_bv, l1_Wo, l1_bo,
           l2_topo_Wself, l2_topo_Wnei, l2_topo_b,
           l2_seq_Wself, l2_seq_Wnei, l2_seq_b,
           l2_Wq, l2_Wk, l2_Wv, l2_bq, l2_bk, l2_bv, l2_Wo, l2_bo):
    src = edge_index[0]
    dst = edge_index[1]
    _c1, _n1, c2, n2 = _prep()(src, dst)

    y1, cnt = _agg_l1()(x, c2, n2)
    y1 = y1.reshape(NPAD, D_IN)
    w1self = jnp.concatenate([l1_topo_Wself, l1_seq_Wself], axis=1)
    w1nei = jnp.concatenate([l1_topo_Wnei, l1_seq_Wnei], axis=1)
    b1cat = jnp.concatenate([l1_topo_b, l1_seq_b])[None, :]
    h = _dense_l1(x, y1[:N], cnt[:N], w1self, w1nei, b1cat,
                  l1_Wq, l1_Wk, l1_Wv, l1_Wo,
                  l1_bq[None, :], l1_bk[None, :], l1_bv[None, :],
                  l1_bo[None, :])

    y2 = _agg_l2()(h, c2, n2)
    if isinstance(y2, (list, tuple)):
        y2, = y2
    y2 = y2.reshape(NPAD, D_HID)
    w2self = jnp.concatenate([l2_topo_Wself, l2_seq_Wself], axis=1)
    w2nei = jnp.concatenate([l2_topo_Wnei, l2_seq_Wnei], axis=1)
    b2cat = jnp.concatenate([l2_topo_b, l2_seq_b])[None, :]
    out = _dense_l2(h, y2[:N], cnt[:N], w2self, w2nei, b2cat,
                    l2_Wq, l2_Wk, l2_Wv, l2_Wo,
                    l2_bq[None, :], l2_bk[None, :], l2_bv[None, :],
                    l2_bo[None, :])
    return out
